# Initial kernel scaffold; baseline (speedup 1.0000x reference)
import jax
import jax.numpy as jnp
from jax.experimental import pallas as pl


def kernel(emb_user, emb_item, edge_weight, user_conf, item_pop, weight, bias, edge_index, users):
    raise NotImplementedError("write your pallas kernel here")


# speedup vs baseline: 5.0311x; 5.0311x over previous
import jax
import jax.numpy as jnp
from jax import lax
from jax.experimental import pallas as pl
from jax.experimental.pallas import tpu as pltpu
from jax.experimental.pallas import tpu_sc as plsc

NU = 25000
NI = 25000
NN = NU + NI
D = 64
DH = 32
NLAY = 3
E = 800000
BQ = 1024

NC = 2
NS = 16
CHUNK = 128
BR = 16
NB = 25
RPT = BR * NB
EROWS = RPT * NS
EPAD = EROWS * CHUNK

RZT = 3128
MCH = 100
NMC = NI // MCH
NZ = 32
UPT = BQ // NS


def _sc_body(t0, srcr, dstr, wr, uall,
             t1o, t2o, t3o, imean, uemb,
             acc, ebs, ebd, ebw, rows, tbuf, ob, ubuf, us, uidx, sem):
  c = lax.axis_index("c")
  s = lax.axis_index("s")

  def zfill(i, _):
    ob[i, pl.ds(0, 16)] = jnp.zeros((16,), jnp.float32)
    ob[i, pl.ds(16, 16)] = jnp.zeros((16,), jnp.float32)
    return 0
  lax.fori_loop(0, MCH, zfill, 0)
  zb = jnp.minimum(s * RZT, NN - RZT)

  tins = (t0, t1o, t2o)
  touts = (t1o, t2o, t3o)
  for t in range(NLAY):
    tin = tins[t]
    tout = touts[t]
    for k in range(NZ):
      off = jnp.minimum(zb + k * MCH, NN - MCH)
      pltpu.sync_copy(ob, acc.at[pl.ds(off, MCH)])
    plsc.subcore_barrier()

    def block(b, _):
      row0 = s * RPT + b * BR
      pltpu.sync_copy(srcr.at[pl.ds(c * EROWS + row0, BR)], ebs)
      pltpu.sync_copy(dstr.at[pl.ds(row0, BR)], ebd)
      pltpu.sync_copy(wr.at[pl.ds(row0, BR)], ebw)

      def chunk(j, _):
        pltpu.async_copy(tin.at[ebs.at[j]], rows, sem).wait()

        def group(g, _):
          wg = ebw[j, pl.ds(g * 16, 16)]
          for l in range(16):
            e = g * 16 + l
            wv = jnp.full((16,), wg[l], jnp.float32)
            rows[e, pl.ds(0, 16)] = rows[e, pl.ds(0, 16)] * wv
            rows[e, pl.ds(16, 16)] = rows[e, pl.ds(16, 16)] * wv
          return 0
        lax.fori_loop(0, 8, group, 0)
        pltpu.sync_copy(rows, acc.at[ebd.at[j]], add=True)
        return 0
      lax.fori_loop(0, BR, chunk, 0)
      return 0
    lax.fori_loop(0, NB, block, 0)
    plsc.subcore_barrier()
    pltpu.sync_copy(acc.at[pl.ds(zb, RZT)],
                    tout.at[pl.ds(c * NN, NN)].at[pl.ds(zb, RZT)])
    plsc.subcore_barrier()

  def mean_chunk(k, _):
    ck = s + NS * k

    @pl.when(ck < NMC)
    def _():
      base = c * NN + NU + ck * MCH
      for ti, tref in enumerate((t0, t1o, t2o, t3o)):
        pltpu.sync_copy(tref.at[pl.ds(base, MCH)], tbuf)

        def row(i, _):
          for h in range(2):
            sl = pl.ds(h * 16, 16)
            if ti == 0:
              ob[i, sl] = tbuf[i, sl]
            elif ti < NLAY:
              ob[i, sl] = ob[i, sl] + tbuf[i, sl]
            else:
              ob[i, sl] = (ob[i, sl] + tbuf[i, sl]) * 0.25
          return 0
        lax.fori_loop(0, MCH, row, 0)
      pltpu.sync_copy(ob, imean.at[pl.ds(c * NI + ck * MCH, MCH)])
    return 0
  lax.fori_loop(0, (NMC + NS - 1) // NS, mean_chunk, 0)

  ub = s * UPT
  pltpu.sync_copy(uall.at[pl.ds(c * BQ + ub, UPT)], uidx)
  for ti, tref in enumerate((t0, t1o, t2o, t3o)):
    pltpu.async_copy(tref.at[uidx], ubuf, sem).wait()

    def urow(i, _):
      for h in range(2):
        sl = pl.ds(h * 16, 16)
        if ti == 0:
          us[i, sl] = ubuf[i, sl]
        elif ti < NLAY:
          us[i, sl] = us[i, sl] + ubuf[i, sl]
        else:
          us[i, sl] = (us[i, sl] + ubuf[i, sl]) * 0.25
      return 0
    lax.fori_loop(0, UPT, urow, 0)
  pltpu.sync_copy(us, uemb.at[pl.ds(c * BQ + ub, UPT)])


def _run_sc(t0, srcr, dstr, wr, uall):
  mesh = plsc.VectorSubcoreMesh(core_axis_name="c", subcore_axis_name="s",
                                num_cores=NC, num_subcores=NS)
  f = pl.kernel(
      _sc_body,
      out_type=(
          jax.ShapeDtypeStruct((NC * NN, DH), jnp.float32),
          jax.ShapeDtypeStruct((NC * NN, DH), jnp.float32),
          jax.ShapeDtypeStruct((NC * NN, DH), jnp.float32),
          jax.ShapeDtypeStruct((NC * NI, DH), jnp.float32),
          jax.ShapeDtypeStruct((NC * BQ, DH), jnp.float32),
      ),
      mesh=mesh,
      scratch_types=[
          pltpu.VMEM_SHARED((NN, DH), jnp.float32),
          pltpu.VMEM((BR, CHUNK), jnp.int32),
          pltpu.VMEM((BR, CHUNK), jnp.int32),
          pltpu.VMEM((BR, CHUNK), jnp.float32),
          pltpu.VMEM((CHUNK, DH), jnp.float32),
          pltpu.VMEM((MCH, DH), jnp.float32),
          pltpu.VMEM((MCH, DH), jnp.float32),
          pltpu.VMEM((UPT, DH), jnp.float32),
          pltpu.VMEM((UPT, DH), jnp.float32),
          pltpu.VMEM((UPT,), jnp.int32),
          pltpu.SemaphoreType.DMA,
      ],
      compiler_params=pltpu.CompilerParams(use_tc_tiling_on_sc=False),
  )
  return f(t0, srcr, dstr, wr, uall)


IB = 512
NIB = pl.cdiv(NI, IB)


def _tc_body(ue_ref, ia_ref, ib_ref, uc_ref, ip_ref, wb_ref, rating_ref, orig_ref):
  ue = ue_ref[...]
  dn = (((1,), (1,)), ((), ()))
  logits = lax.dot_general(ue[:, :DH], ia_ref[...], dn,
                           preferred_element_type=jnp.float32)
  logits += lax.dot_general(ue[:, DH:], ib_ref[...], dn,
                            preferred_element_type=jnp.float32)
  orig_ref[...] = jax.nn.sigmoid(logits)
  adj = uc_ref[...] * ip_ref[...] * wb_ref[0, 0] + wb_ref[0, 1]
  rating_ref[...] = jax.nn.sigmoid(logits + adj)


def _run_tc(ue, ia, ib, uc, ip, wb):
  return pl.pallas_call(
      _tc_body,
      grid=(NIB,),
      in_specs=[
          pl.BlockSpec((BQ, D), lambda i: (0, 0)),
          pl.BlockSpec((IB, DH), lambda i: (i, 0)),
          pl.BlockSpec((IB, DH), lambda i: (i, 0)),
          pl.BlockSpec((BQ, 1), lambda i: (0, 0)),
          pl.BlockSpec((1, IB), lambda i: (0, i)),
          pl.BlockSpec(memory_space=pltpu.SMEM),
      ],
      out_specs=[
          pl.BlockSpec((BQ, IB), lambda i: (0, i)),
          pl.BlockSpec((BQ, IB), lambda i: (0, i)),
      ],
      out_shape=[
          jax.ShapeDtypeStruct((BQ, NI), jnp.float32),
          jax.ShapeDtypeStruct((BQ, NI), jnp.float32),
      ],
  )(ue, ia, ib, uc, ip, wb)


def kernel(emb_user, emb_item, edge_weight, user_conf, item_pop, weight, bias,
           edge_index, users):
  allemb = jnp.concatenate([emb_user, emb_item], axis=0)
  t0 = allemb.reshape(NN, NC, DH).transpose(1, 0, 2).reshape(NC * NN, DH)

  src = edge_index[0].astype(jnp.int32)
  dst = edge_index[1].astype(jnp.int32)
  w = edge_weight.astype(jnp.float32)
  pad = EPAD - E
  spread = jnp.arange(pad, dtype=jnp.int32) % NN
  srcp = jnp.concatenate([src, spread])
  srcr = jnp.concatenate([srcp, srcp + NN]).reshape(NC * EROWS, CHUNK)
  dstr = jnp.concatenate([dst, spread]).reshape(EROWS, CHUNK)
  wr = jnp.pad(w, (0, pad)).reshape(EROWS, CHUNK)

  users_i = users.astype(jnp.int32)
  uall = jnp.concatenate([users_i, users_i + NN])

  t1o, t2o, t3o, imean, uemb = _run_sc(t0, srcr, dstr, wr, uall)

  ue = uemb.reshape(NC, BQ, DH).transpose(1, 0, 2).reshape(BQ, D)
  uc = user_conf[users_i, 0].astype(jnp.float32).reshape(BQ, 1)
  ip = item_pop.astype(jnp.float32).reshape(1, NI)
  wb = jnp.stack([weight.astype(jnp.float32),
                  bias.astype(jnp.float32)]).reshape(1, 2)
  rating, rating_orig = _run_tc(ue, imean[:NI], imean[NI:], uc, ip, wb)
  return (rating, rating_orig)


# speedup vs baseline: 6.4328x; 1.2786x over previous
import jax
import jax.numpy as jnp
from jax import lax
from jax.experimental import pallas as pl
from jax.experimental.pallas import tpu as pltpu
from jax.experimental.pallas import tpu_sc as plsc

NU = 25000
NI = 25000
NN = NU + NI
D = 64
DH = 32
NLAY = 3
E = 800000
BQ = 1024

NC = 2
NS = 16
CHUNK = 128
BR = 16
NB = 25
RPT = BR * NB
EROWS = RPT * NS
EPAD = EROWS * CHUNK

RZT = 3128
MCH = 100
NMC = NI // MCH
NZ = 32
UPT = BQ // NS


def _sc_body(t0, srcr, dstr, wr, uall,
             t1o, t2o, t3o, imean, uemb,
             acc, ebs, ebd, ebw, rows0, rows1, tbuf, ob, ubuf, us, uidx,
             sem, gsem0, gsem1):
  c = lax.axis_index("c")
  s = lax.axis_index("s")

  def zfill(i, _):
    ob[i, pl.ds(0, 16)] = jnp.zeros((16,), jnp.float32)
    ob[i, pl.ds(16, 16)] = jnp.zeros((16,), jnp.float32)
    return 0
  lax.fori_loop(0, MCH, zfill, 0)
  zb = jnp.minimum(s * RZT, NN - RZT)

  tins = (t0, t1o, t2o)
  touts = (t1o, t2o, t3o)
  for t in range(NLAY):
    tin = tins[t]
    tout = touts[t]
    for k in range(NZ):
      off = jnp.minimum(zb + k * MCH, NN - MCH)
      pltpu.async_copy(ob, acc.at[pl.ds(off, MCH)], sem)
    for k in range(NZ):
      off = jnp.minimum(zb + k * MCH, NN - MCH)
      pltpu.make_async_copy(ob, acc.at[pl.ds(off, MCH)], sem).wait()
    plsc.subcore_barrier()

    def scale(rbuf, j):
      def group(g, _):
        wg = ebw[j, pl.ds(g * 16, 16)]
        for l in range(16):
          e = g * 16 + l
          wv = jnp.full((16,), wg[l], jnp.float32)
          rbuf[e, pl.ds(0, 16)] = rbuf[e, pl.ds(0, 16)] * wv
          rbuf[e, pl.ds(16, 16)] = rbuf[e, pl.ds(16, 16)] * wv
        return 0
      lax.fori_loop(0, 8, group, 0)

    def block(b, _):
      row0 = s * RPT + b * BR
      pltpu.sync_copy(srcr.at[pl.ds(c * EROWS + row0, BR)], ebs)
      pltpu.sync_copy(dstr.at[pl.ds(row0, BR)], ebd)
      pltpu.sync_copy(wr.at[pl.ds(row0, BR)], ebw)

      pltpu.async_copy(tin.at[ebs.at[0]], rows0, gsem0)

      def pair(p, _):
        j0 = 2 * p
        j1 = 2 * p + 1
        pltpu.make_async_copy(tin.at[ebs.at[j0]], rows0, gsem0).wait()
        pltpu.async_copy(tin.at[ebs.at[j1]], rows1, gsem1)
        scale(rows0, j0)
        pltpu.sync_copy(rows0, acc.at[ebd.at[j0]], add=True)
        pltpu.make_async_copy(tin.at[ebs.at[j1]], rows1, gsem1).wait()

        @pl.when(p + 1 < BR // 2)
        def _():
          pltpu.async_copy(tin.at[ebs.at[j0 + 2]], rows0, gsem0)
        scale(rows1, j1)
        pltpu.sync_copy(rows1, acc.at[ebd.at[j1]], add=True)
        return 0
      lax.fori_loop(0, BR // 2, pair, 0)
      return 0
    lax.fori_loop(0, NB, block, 0)
    plsc.subcore_barrier()
    pltpu.sync_copy(acc.at[pl.ds(zb, RZT)],
                    tout.at[pl.ds(c * NN, NN)].at[pl.ds(zb, RZT)])
    plsc.subcore_barrier()

  def mean_chunk(k, _):
    ck = s + NS * k

    @pl.when(ck < NMC)
    def _():
      base = c * NN + NU + ck * MCH
      for ti, tref in enumerate((t0, t1o, t2o, t3o)):
        pltpu.sync_copy(tref.at[pl.ds(base, MCH)], tbuf)

        def row(i, _):
          for h in range(2):
            sl = pl.ds(h * 16, 16)
            if ti == 0:
              ob[i, sl] = tbuf[i, sl]
            elif ti < NLAY:
              ob[i, sl] = ob[i, sl] + tbuf[i, sl]
            else:
              ob[i, sl] = (ob[i, sl] + tbuf[i, sl]) * 0.25
          return 0
        lax.fori_loop(0, MCH, row, 0)
      pltpu.sync_copy(ob, imean.at[pl.ds(c * NI + ck * MCH, MCH)])
    return 0
  lax.fori_loop(0, (NMC + NS - 1) // NS, mean_chunk, 0)

  ub = s * UPT
  pltpu.sync_copy(uall.at[pl.ds(c * BQ + ub, UPT)], uidx)
  for ti, tref in enumerate((t0, t1o, t2o, t3o)):
    pltpu.async_copy(tref.at[uidx], ubuf, sem).wait()

    def urow(i, _):
      for h in range(2):
        sl = pl.ds(h * 16, 16)
        if ti == 0:
          us[i, sl] = ubuf[i, sl]
        elif ti < NLAY:
          us[i, sl] = us[i, sl] + ubuf[i, sl]
        else:
          us[i, sl] = (us[i, sl] + ubuf[i, sl]) * 0.25
      return 0
    lax.fori_loop(0, UPT, urow, 0)
  pltpu.sync_copy(us, uemb.at[pl.ds(c * BQ + ub, UPT)])


def _run_sc(t0, srcr, dstr, wr, uall):
  mesh = plsc.VectorSubcoreMesh(core_axis_name="c", subcore_axis_name="s",
                                num_cores=NC, num_subcores=NS)
  f = pl.kernel(
      _sc_body,
      out_type=(
          jax.ShapeDtypeStruct((NC * NN, DH), jnp.float32),
          jax.ShapeDtypeStruct((NC * NN, DH), jnp.float32),
          jax.ShapeDtypeStruct((NC * NN, DH), jnp.float32),
          jax.ShapeDtypeStruct((NC * NI, DH), jnp.float32),
          jax.ShapeDtypeStruct((NC * BQ, DH), jnp.float32),
      ),
      mesh=mesh,
      scratch_types=[
          pltpu.VMEM_SHARED((NN, DH), jnp.float32),
          pltpu.VMEM((BR, CHUNK), jnp.int32),
          pltpu.VMEM((BR, CHUNK), jnp.int32),
          pltpu.VMEM((BR, CHUNK), jnp.float32),
          pltpu.VMEM((CHUNK, DH), jnp.float32),
          pltpu.VMEM((CHUNK, DH), jnp.float32),
          pltpu.VMEM((MCH, DH), jnp.float32),
          pltpu.VMEM((MCH, DH), jnp.float32),
          pltpu.VMEM((UPT, DH), jnp.float32),
          pltpu.VMEM((UPT, DH), jnp.float32),
          pltpu.VMEM((UPT,), jnp.int32),
          pltpu.SemaphoreType.DMA,
          pltpu.SemaphoreType.DMA,
          pltpu.SemaphoreType.DMA,
      ],
      compiler_params=pltpu.CompilerParams(use_tc_tiling_on_sc=False),
  )
  return f(t0, srcr, dstr, wr, uall)


IB = 512
NIB = pl.cdiv(NI, IB)


def _tc_body(ue_ref, ia_ref, ib_ref, uc_ref, ip_ref, wb_ref, rating_ref, orig_ref):
  ue = ue_ref[...]
  dn = (((1,), (1,)), ((), ()))
  logits = lax.dot_general(ue[:, :DH], ia_ref[...], dn,
                           preferred_element_type=jnp.float32)
  logits += lax.dot_general(ue[:, DH:], ib_ref[...], dn,
                            preferred_element_type=jnp.float32)
  orig_ref[...] = jax.nn.sigmoid(logits)
  adj = uc_ref[...] * ip_ref[...] * wb_ref[0, 0] + wb_ref[0, 1]
  rating_ref[...] = jax.nn.sigmoid(logits + adj)


def _run_tc(ue, ia, ib, uc, ip, wb):
  return pl.pallas_call(
      _tc_body,
      grid=(NIB,),
      in_specs=[
          pl.BlockSpec((BQ, D), lambda i: (0, 0)),
          pl.BlockSpec((IB, DH), lambda i: (i, 0)),
          pl.BlockSpec((IB, DH), lambda i: (i, 0)),
          pl.BlockSpec((BQ, 1), lambda i: (0, 0)),
          pl.BlockSpec((1, IB), lambda i: (0, i)),
          pl.BlockSpec(memory_space=pltpu.SMEM),
      ],
      out_specs=[
          pl.BlockSpec((BQ, IB), lambda i: (0, i)),
          pl.BlockSpec((BQ, IB), lambda i: (0, i)),
      ],
      out_shape=[
          jax.ShapeDtypeStruct((BQ, NI), jnp.float32),
          jax.ShapeDtypeStruct((BQ, NI), jnp.float32),
      ],
  )(ue, ia, ib, uc, ip, wb)


def kernel(emb_user, emb_item, edge_weight, user_conf, item_pop, weight, bias,
           edge_index, users):
  allemb = jnp.concatenate([emb_user, emb_item], axis=0)
  t0 = allemb.reshape(NN, NC, DH).transpose(1, 0, 2).reshape(NC * NN, DH)

  src = edge_index[0].astype(jnp.int32)
  dst = edge_index[1].astype(jnp.int32)
  w = edge_weight.astype(jnp.float32)
  pad = EPAD - E
  spread = jnp.arange(pad, dtype=jnp.int32) % NN
  srcp = jnp.concatenate([src, spread])
  srcr = jnp.concatenate([srcp, srcp + NN]).reshape(NC * EROWS, CHUNK)
  dstr = jnp.concatenate([dst, spread]).reshape(EROWS, CHUNK)
  wr = jnp.pad(w, (0, pad)).reshape(EROWS, CHUNK)

  users_i = users.astype(jnp.int32)
  uall = jnp.concatenate([users_i, users_i + NN])

  t1o, t2o, t3o, imean, uemb = _run_sc(t0, srcr, dstr, wr, uall)

  ue = uemb.reshape(NC, BQ, DH).transpose(1, 0, 2).reshape(BQ, D)
  uc = user_conf[users_i, 0].astype(jnp.float32).reshape(BQ, 1)
  ip = item_pop.astype(jnp.float32).reshape(1, NI)
  wb = jnp.stack([weight.astype(jnp.float32),
                  bias.astype(jnp.float32)]).reshape(1, 2)
  rating, rating_orig = _run_tc(ue, imean[:NI], imean[NI:], uc, ip, wb)
  return (rating, rating_orig)


# speedup vs baseline: 9.2515x; 1.4382x over previous
import jax
import jax.numpy as jnp
from jax import lax
from jax.experimental import pallas as pl
from jax.experimental.pallas import tpu as pltpu
from jax.experimental.pallas import tpu_sc as plsc

NU = 25000
NI = 25000
NN = NU + NI
D = 64
DH = 32
NLAY = 3
E = 800000
BQ = 1024

NC = 2
NS = 16
CHUNK = 128
BR = 16
NB = 25
RPT = BR * NB
EROWS = RPT * NS
EPAD = EROWS * CHUNK

RZT = 3128
NZ = 25
MCH = 125
NMC = NI // MCH
UPT = BQ // NS


def _sc_body(t0, srcr, dstr, wr, uall,
             t1o, t2o, t3o, imean, uemb,
             acc, ebs, ebd, ebw, rows0, rows1, rows2, rows3, ubuf, us, uidx,
             sem, g0, g1, g2, g3, s0, s1, s2, s3):
  c = lax.axis_index("c")
  s = lax.axis_index("s")
  rbufs = (rows0, rows1, rows2, rows3)
  gsems = (g0, g1, g2, g3)
  ssems = (s0, s1, s2, s3)
  zb = jnp.minimum(s * RZT, NN - RZT)

  def scale(rbuf, j):
    def group(g, _):
      wg = ebw[j, pl.ds(g * 16, 16)]
      for l in range(16):
        e = g * 16 + l
        wv = jnp.full((16,), wg[l], jnp.float32)
        rbuf[e, pl.ds(0, 16)] = rbuf[e, pl.ds(0, 16)] * wv
        rbuf[e, pl.ds(16, 16)] = rbuf[e, pl.ds(16, 16)] * wv
      return 0
    lax.fori_loop(0, 8, group, 0)

  tins = (t0, t1o, t2o)
  touts = (t1o, t2o, t3o)
  for t in range(NLAY):
    tin = tins[t]
    tout = touts[t]
    def zfill(i, _):
      rows2[i, pl.ds(0, 16)] = jnp.zeros((16,), jnp.float32)
      rows2[i, pl.ds(16, 16)] = jnp.zeros((16,), jnp.float32)
      return 0
    lax.fori_loop(0, CHUNK, zfill, 0)
    for k in range(NZ):
      off = jnp.minimum(zb + k * CHUNK, NN - CHUNK)
      pltpu.async_copy(rows2, acc.at[pl.ds(off, CHUNK)], sem)
    for k in range(NZ):
      off = jnp.minimum(zb + k * CHUNK, NN - CHUNK)
      pltpu.make_async_copy(rows2, acc.at[pl.ds(off, CHUNK)], sem).wait()
    plsc.subcore_barrier()

    def block(b, _):
      row0 = s * RPT + b * BR
      pltpu.async_copy(srcr.at[pl.ds(c * EROWS + row0, BR)], ebs, sem)
      pltpu.async_copy(dstr.at[pl.ds(row0, BR)], ebd, sem)
      pltpu.async_copy(wr.at[pl.ds(row0, BR)], ebw, sem)
      pltpu.make_async_copy(srcr.at[pl.ds(c * EROWS + row0, BR)], ebs, sem).wait()
      pltpu.make_async_copy(dstr.at[pl.ds(row0, BR)], ebd, sem).wait()
      pltpu.make_async_copy(wr.at[pl.ds(row0, BR)], ebw, sem).wait()

      pltpu.async_copy(tin.at[ebs.at[0]], rows0, g0)
      pltpu.async_copy(tin.at[ebs.at[1]], rows1, g1)

      def quad(q, _):
        for k in range(4):
          j = 4 * q + k
          rb = rbufs[k]
          pltpu.make_async_copy(tin.at[ebs.at[j]], rb, gsems[k]).wait()
          kp = (k + 2) % 4
          if k < 2:
            @pl.when(q > 0)
            def _():
              pltpu.make_async_copy(rbufs[kp], acc.at[ebd.at[j]],
                                    ssems[kp]).wait()
            pltpu.async_copy(tin.at[ebs.at[j + 2]], rbufs[kp], gsems[kp])
          else:
            pltpu.make_async_copy(rbufs[kp], acc.at[ebd.at[j]],
                                  ssems[kp]).wait()

            @pl.when(j + 2 < BR)
            def _():
              pltpu.async_copy(tin.at[ebs.at[j + 2]], rbufs[kp], gsems[kp])
          scale(rb, j)
          pltpu.async_copy(rb, acc.at[ebd.at[j]], ssems[k], add=True)
        return 0
      lax.fori_loop(0, BR // 4, quad, 0)
      for k in (2, 3):
        pltpu.make_async_copy(rbufs[k], acc.at[ebd.at[0]], ssems[k]).wait()
      return 0
    lax.fori_loop(0, NB, block, 0)
    plsc.subcore_barrier()
    pltpu.sync_copy(acc.at[pl.ds(zb, RZT)],
                    tout.at[pl.ds(c * NN, NN)].at[pl.ds(zb, RZT)])
    plsc.subcore_barrier()

  def mean_chunk(k, _):
    ck = s + NS * k

    @pl.when(ck < NMC)
    def _():
      base = c * NN + NU + ck * MCH
      for ti, tref in enumerate((t0, t1o, t2o, t3o)):
        pltpu.sync_copy(tref.at[pl.ds(base, MCH)], rows0.at[pl.ds(0, MCH)])

        def row(i, _):
          for h in range(2):
            sl = pl.ds(h * 16, 16)
            if ti == 0:
              rows1[i, sl] = rows0[i, sl]
            elif ti < NLAY:
              rows1[i, sl] = rows1[i, sl] + rows0[i, sl]
            else:
              rows1[i, sl] = (rows1[i, sl] + rows0[i, sl]) * 0.25
          return 0
        lax.fori_loop(0, MCH, row, 0)
      pltpu.sync_copy(rows1.at[pl.ds(0, MCH)],
                      imean.at[pl.ds(c * NI + ck * MCH, MCH)])
    return 0
  lax.fori_loop(0, (NMC + NS - 1) // NS, mean_chunk, 0)

  ub = s * UPT
  pltpu.sync_copy(uall.at[pl.ds(c * BQ + ub, UPT)], uidx)
  for ti, tref in enumerate((t0, t1o, t2o, t3o)):
    pltpu.async_copy(tref.at[uidx], ubuf, sem).wait()

    def urow(i, _):
      for h in range(2):
        sl = pl.ds(h * 16, 16)
        if ti == 0:
          us[i, sl] = ubuf[i, sl]
        elif ti < NLAY:
          us[i, sl] = us[i, sl] + ubuf[i, sl]
        else:
          us[i, sl] = (us[i, sl] + ubuf[i, sl]) * 0.25
      return 0
    lax.fori_loop(0, UPT, urow, 0)
  pltpu.sync_copy(us, uemb.at[pl.ds(c * BQ + ub, UPT)])


def _run_sc(t0, srcr, dstr, wr, uall):
  mesh = plsc.VectorSubcoreMesh(core_axis_name="c", subcore_axis_name="s",
                                num_cores=NC, num_subcores=NS)
  f = pl.kernel(
      _sc_body,
      out_type=(
          jax.ShapeDtypeStruct((NC * NN, DH), jnp.float32),
          jax.ShapeDtypeStruct((NC * NN, DH), jnp.float32),
          jax.ShapeDtypeStruct((NC * NN, DH), jnp.float32),
          jax.ShapeDtypeStruct((NC * NI, DH), jnp.float32),
          jax.ShapeDtypeStruct((NC * BQ, DH), jnp.float32),
      ),
      mesh=mesh,
      scratch_types=[
          pltpu.VMEM_SHARED((NN, DH), jnp.float32),
          pltpu.VMEM((BR, CHUNK), jnp.int32),
          pltpu.VMEM((BR, CHUNK), jnp.int32),
          pltpu.VMEM((BR, CHUNK), jnp.float32),
          pltpu.VMEM((CHUNK, DH), jnp.float32),
          pltpu.VMEM((CHUNK, DH), jnp.float32),
          pltpu.VMEM((CHUNK, DH), jnp.float32),
          pltpu.VMEM((CHUNK, DH), jnp.float32),
          pltpu.VMEM((UPT, DH), jnp.float32),
          pltpu.VMEM((UPT, DH), jnp.float32),
          pltpu.VMEM((UPT,), jnp.int32),
          pltpu.SemaphoreType.DMA,
          pltpu.SemaphoreType.DMA,
          pltpu.SemaphoreType.DMA,
          pltpu.SemaphoreType.DMA,
          pltpu.SemaphoreType.DMA,
          pltpu.SemaphoreType.DMA,
          pltpu.SemaphoreType.DMA,
          pltpu.SemaphoreType.DMA,
          pltpu.SemaphoreType.DMA,
      ],
      compiler_params=pltpu.CompilerParams(use_tc_tiling_on_sc=False),
  )
  return f(t0, srcr, dstr, wr, uall)


IB = 512
NIB = pl.cdiv(NI, IB)


def _tc_body(ue_ref, ia_ref, ib_ref, uc_ref, ip_ref, wb_ref, rating_ref, orig_ref):
  ue = ue_ref[...]
  dn = (((1,), (1,)), ((), ()))
  logits = lax.dot_general(ue[:, :DH], ia_ref[...], dn,
                           preferred_element_type=jnp.float32)
  logits += lax.dot_general(ue[:, DH:], ib_ref[...], dn,
                            preferred_element_type=jnp.float32)
  orig_ref[...] = jax.nn.sigmoid(logits)
  adj = uc_ref[...] * ip_ref[...] * wb_ref[0, 0] + wb_ref[0, 1]
  rating_ref[...] = jax.nn.sigmoid(logits + adj)


def _run_tc(ue, ia, ib, uc, ip, wb):
  return pl.pallas_call(
      _tc_body,
      grid=(NIB,),
      in_specs=[
          pl.BlockSpec((BQ, D), lambda i: (0, 0)),
          pl.BlockSpec((IB, DH), lambda i: (i, 0)),
          pl.BlockSpec((IB, DH), lambda i: (i, 0)),
          pl.BlockSpec((BQ, 1), lambda i: (0, 0)),
          pl.BlockSpec((1, IB), lambda i: (0, i)),
          pl.BlockSpec(memory_space=pltpu.SMEM),
      ],
      out_specs=[
          pl.BlockSpec((BQ, IB), lambda i: (0, i)),
          pl.BlockSpec((BQ, IB), lambda i: (0, i)),
      ],
      out_shape=[
          jax.ShapeDtypeStruct((BQ, NI), jnp.float32),
          jax.ShapeDtypeStruct((BQ, NI), jnp.float32),
      ],
  )(ue, ia, ib, uc, ip, wb)


def kernel(emb_user, emb_item, edge_weight, user_conf, item_pop, weight, bias,
           edge_index, users):
  allemb = jnp.concatenate([emb_user, emb_item], axis=0)
  t0 = allemb.reshape(NN, NC, DH).transpose(1, 0, 2).reshape(NC * NN, DH)

  src = edge_index[0].astype(jnp.int32)
  dst = edge_index[1].astype(jnp.int32)
  w = edge_weight.astype(jnp.float32)
  pad = EPAD - E
  spread = jnp.arange(pad, dtype=jnp.int32) % NN
  srcp = jnp.concatenate([src, spread])
  srcr = jnp.concatenate([srcp, srcp + NN]).reshape(NC * EROWS, CHUNK)
  dstr = jnp.concatenate([dst, spread]).reshape(EROWS, CHUNK)
  wr = jnp.pad(w, (0, pad)).reshape(EROWS, CHUNK)

  users_i = users.astype(jnp.int32)
  uall = jnp.concatenate([users_i, users_i + NN])

  t1o, t2o, t3o, imean, uemb = _run_sc(t0, srcr, dstr, wr, uall)

  ue = uemb.reshape(NC, BQ, DH).transpose(1, 0, 2).reshape(BQ, D)
  uc = user_conf[users_i, 0].astype(jnp.float32).reshape(BQ, 1)
  ip = item_pop.astype(jnp.float32).reshape(1, NI)
  wb = jnp.stack([weight.astype(jnp.float32),
                  bias.astype(jnp.float32)]).reshape(1, 2)
  rating, rating_orig = _run_tc(ue, imean[:NI], imean[NI:], uc, ip, wb)
  return (rating, rating_orig)


# speedup vs baseline: 9.2531x; 1.0002x over previous
import jax
import jax.numpy as jnp
from jax import lax
from jax.experimental import pallas as pl
from jax.experimental.pallas import tpu as pltpu
from jax.experimental.pallas import tpu_sc as plsc

NU = 25000
NI = 25000
NN = NU + NI
D = 64
DH = 32
NLAY = 3
E = 800000
BQ = 1024

NC = 2
NS = 16
CHUNK = 128
BR = 16
NB = 25
RPT = BR * NB
EROWS = RPT * NS
EPAD = EROWS * CHUNK

RZT = 3128
NZ = 25
MCH = 125
NMC = NI // MCH
UPT = BQ // NS


def _sc_body(t0, srcr, dstr, wr, uall,
             t1o, t2o, t3o, imean, uemb,
             acc, ebs, ebd, ebw, rows0, rows1, rows2, rows3, ubuf, us, uidx,
             sem, g0, g1, g2, g3, s0, s1, s2, s3):
  c = lax.axis_index("c")
  s = lax.axis_index("s")
  rbufs = (rows0, rows1, rows2, rows3)
  gsems = (g0, g1, g2, g3)
  ssems = (s0, s1, s2, s3)
  zb = jnp.minimum(s * RZT, NN - RZT)

  def scale(rb, j):
    def grp(g, _):
      wv = ebw[j, pl.ds(g * 16, 16)]
      for k in range(16):
        w = wv[k]
        i = g * 16 + k
        rb[i, pl.ds(0, 16)] = rb[i, pl.ds(0, 16)] * w
        rb[i, pl.ds(16, 16)] = rb[i, pl.ds(16, 16)] * w
      return 0
    lax.fori_loop(0, CHUNK // 16, grp, 0)

  tins = (t0, t1o, t2o)
  touts = (t1o, t2o, t3o)
  for t in range(NLAY):
    tin = tins[t]
    tout = touts[t]
    def zfill(i, _):
      rows2[i, pl.ds(0, 16)] = jnp.zeros((16,), jnp.float32)
      rows2[i, pl.ds(16, 16)] = jnp.zeros((16,), jnp.float32)
      return 0
    lax.fori_loop(0, CHUNK, zfill, 0)
    for k in range(NZ):
      off = jnp.minimum(zb + k * CHUNK, NN - CHUNK)
      pltpu.async_copy(rows2, acc.at[pl.ds(off, CHUNK)], sem)
    for k in range(NZ):
      off = jnp.minimum(zb + k * CHUNK, NN - CHUNK)
      pltpu.make_async_copy(rows2, acc.at[pl.ds(off, CHUNK)], sem).wait()
    plsc.subcore_barrier()

    def block(b, _):
      row0 = s * RPT + b * BR
      pltpu.async_copy(srcr.at[pl.ds(c * EROWS + row0, BR)], ebs, sem)
      pltpu.async_copy(dstr.at[pl.ds(row0, BR)], ebd, sem)
      pltpu.async_copy(wr.at[pl.ds(row0, BR)], ebw, sem)
      pltpu.make_async_copy(srcr.at[pl.ds(c * EROWS + row0, BR)], ebs, sem).wait()
      pltpu.make_async_copy(dstr.at[pl.ds(row0, BR)], ebd, sem).wait()
      pltpu.make_async_copy(wr.at[pl.ds(row0, BR)], ebw, sem).wait()

      pltpu.async_copy(tin.at[ebs.at[0]], rows0, g0)
      pltpu.async_copy(tin.at[ebs.at[1]], rows1, g1)

      def quad(q, _):
        for k in range(4):
          j = 4 * q + k
          rb = rbufs[k]
          pltpu.make_async_copy(tin.at[ebs.at[j]], rb, gsems[k]).wait()
          kp = (k + 2) % 4
          if k < 2:
            @pl.when(q > 0)
            def _():
              pltpu.make_async_copy(rbufs[kp], acc.at[ebd.at[j]],
                                    ssems[kp]).wait()
            pltpu.async_copy(tin.at[ebs.at[j + 2]], rbufs[kp], gsems[kp])
          else:
            pltpu.make_async_copy(rbufs[kp], acc.at[ebd.at[j]],
                                  ssems[kp]).wait()

            @pl.when(j + 2 < BR)
            def _():
              pltpu.async_copy(tin.at[ebs.at[j + 2]], rbufs[kp], gsems[kp])
          scale(rb, j)
          pltpu.async_copy(rb, acc.at[ebd.at[j]], ssems[k], add=True)
        return 0
      lax.fori_loop(0, BR // 4, quad, 0)
      for k in (2, 3):
        pltpu.make_async_copy(rbufs[k], acc.at[ebd.at[0]], ssems[k]).wait()
      return 0
    lax.fori_loop(0, NB, block, 0)
    plsc.subcore_barrier()
    pltpu.sync_copy(acc.at[pl.ds(zb, RZT)],
                    tout.at[pl.ds(c * NN, NN)].at[pl.ds(zb, RZT)])
    plsc.subcore_barrier()

  def mean_chunk(k, _):
    ck = s + NS * k

    @pl.when(ck < NMC)
    def _():
      base = c * NN + NU + ck * MCH
      for ti, tref in enumerate((t0, t1o, t2o, t3o)):
        pltpu.sync_copy(tref.at[pl.ds(base, MCH)], rows0.at[pl.ds(0, MCH)])

        def row(i, _):
          for h in range(2):
            sl = pl.ds(h * 16, 16)
            if ti == 0:
              rows1[i, sl] = rows0[i, sl]
            elif ti < NLAY:
              rows1[i, sl] = rows1[i, sl] + rows0[i, sl]
            else:
              rows1[i, sl] = (rows1[i, sl] + rows0[i, sl]) * 0.25
          return 0
        lax.fori_loop(0, MCH, row, 0)
      pltpu.sync_copy(rows1.at[pl.ds(0, MCH)],
                      imean.at[pl.ds(c * NI + ck * MCH, MCH)])
    return 0
  lax.fori_loop(0, (NMC + NS - 1) // NS, mean_chunk, 0)

  ub = s * UPT
  pltpu.sync_copy(uall.at[pl.ds(c * BQ + ub, UPT)], uidx)
  for ti, tref in enumerate((t0, t1o, t2o, t3o)):
    pltpu.async_copy(tref.at[uidx], ubuf, sem)
    pltpu.make_async_copy(tref.at[uidx], ubuf, sem).wait()

    def urow(i, _):
      for h in range(2):
        sl = pl.ds(h * 16, 16)
        if ti == 0:
          us[i, sl] = ubuf[i, sl]
        elif ti < NLAY:
          us[i, sl] = us[i, sl] + ubuf[i, sl]
        else:
          us[i, sl] = (us[i, sl] + ubuf[i, sl]) * 0.25
      return 0
    lax.fori_loop(0, UPT, urow, 0)
  pltpu.sync_copy(us, uemb.at[pl.ds(c * BQ + ub, UPT)])


def _run_sc(t0, srcr, dstr, wr, uall):
  mesh = plsc.VectorSubcoreMesh(core_axis_name="c", subcore_axis_name="s",
                                num_cores=NC, num_subcores=NS)
  f = pl.kernel(
      _sc_body,
      out_type=(
          jax.ShapeDtypeStruct((NC * NN, DH), jnp.float32),
          jax.ShapeDtypeStruct((NC * NN, DH), jnp.float32),
          jax.ShapeDtypeStruct((NC * NN, DH), jnp.float32),
          jax.ShapeDtypeStruct((NC * NI, DH), jnp.float32),
          jax.ShapeDtypeStruct((NC * BQ, DH), jnp.float32),
      ),
      mesh=mesh,
      scratch_types=[
          pltpu.VMEM_SHARED((NN, DH), jnp.float32),
          pltpu.VMEM((BR, CHUNK), jnp.int32),
          pltpu.VMEM((BR, CHUNK), jnp.int32),
          pltpu.VMEM((BR, CHUNK), jnp.float32),
          pltpu.VMEM((CHUNK, DH), jnp.float32),
          pltpu.VMEM((CHUNK, DH), jnp.float32),
          pltpu.VMEM((CHUNK, DH), jnp.float32),
          pltpu.VMEM((CHUNK, DH), jnp.float32),
          pltpu.VMEM((UPT, DH), jnp.float32),
          pltpu.VMEM((UPT, DH), jnp.float32),
          pltpu.VMEM((UPT,), jnp.int32),
          pltpu.SemaphoreType.DMA,
          pltpu.SemaphoreType.DMA,
          pltpu.SemaphoreType.DMA,
          pltpu.SemaphoreType.DMA,
          pltpu.SemaphoreType.DMA,
          pltpu.SemaphoreType.DMA,
          pltpu.SemaphoreType.DMA,
          pltpu.SemaphoreType.DMA,
          pltpu.SemaphoreType.DMA,
      ],
      compiler_params=pltpu.CompilerParams(use_tc_tiling_on_sc=False),
  )
  return f(t0, srcr, dstr, wr, uall)


IB = 512
NIB = pl.cdiv(NI, IB)


def _tc_body(ue_ref, ia_ref, ib_ref, uc_ref, ip_ref, wb_ref, rating_ref, orig_ref):
  ue = ue_ref[...]
  dn = (((1,), (1,)), ((), ()))
  logits = lax.dot_general(ue[:, :DH], ia_ref[...], dn,
                           preferred_element_type=jnp.float32)
  logits += lax.dot_general(ue[:, DH:], ib_ref[...], dn,
                            preferred_element_type=jnp.float32)
  orig_ref[...] = jax.nn.sigmoid(logits)
  adj = uc_ref[...] * ip_ref[...] * wb_ref[0, 0] + wb_ref[0, 1]
  rating_ref[...] = jax.nn.sigmoid(logits + adj)


def _run_tc(ue, ia, ib, uc, ip, wb):
  return pl.pallas_call(
      _tc_body,
      grid=(NIB,),
      in_specs=[
          pl.BlockSpec((BQ, D), lambda i: (0, 0)),
          pl.BlockSpec((IB, DH), lambda i: (i, 0)),
          pl.BlockSpec((IB, DH), lambda i: (i, 0)),
          pl.BlockSpec((BQ, 1), lambda i: (0, 0)),
          pl.BlockSpec((1, IB), lambda i: (0, i)),
          pl.BlockSpec(memory_space=pltpu.SMEM),
      ],
      out_specs=[
          pl.BlockSpec((BQ, IB), lambda i: (0, i)),
          pl.BlockSpec((BQ, IB), lambda i: (0, i)),
      ],
      out_shape=[
          jax.ShapeDtypeStruct((BQ, NI), jnp.float32),
          jax.ShapeDtypeStruct((BQ, NI), jnp.float32),
      ],
  )(ue, ia, ib, uc, ip, wb)


def kernel(emb_user, emb_item, edge_weight, user_conf, item_pop, weight, bias,
           edge_index, users):
  allemb = jnp.concatenate([emb_user, emb_item], axis=0)
  t0 = allemb.reshape(NN, NC, DH).transpose(1, 0, 2).reshape(NC * NN, DH)

  src = edge_index[0].astype(jnp.int32)
  dst = edge_index[1].astype(jnp.int32)
  w = edge_weight.astype(jnp.float32)
  pad = EPAD - E
  spread = jnp.arange(pad, dtype=jnp.int32) % NN
  srcp = jnp.concatenate([src, spread])
  srcr = jnp.concatenate([srcp, srcp + NN]).reshape(NC * EROWS, CHUNK)
  dstr = jnp.concatenate([dst, spread]).reshape(EROWS, CHUNK)
  wr = jnp.pad(w, (0, pad)).reshape(EROWS, CHUNK)

  users_i = users.astype(jnp.int32)
  uall = jnp.concatenate([users_i, users_i + NN])

  t1o, t2o, t3o, imean, uemb = _run_sc(t0, srcr, dstr, wr, uall)

  ue = uemb.reshape(NC, BQ, DH).transpose(1, 0, 2).reshape(BQ, D)
  uc = user_conf[users_i, 0].astype(jnp.float32).reshape(BQ, 1)
  ip = item_pop.astype(jnp.float32).reshape(1, NI)
  wb = jnp.stack([weight.astype(jnp.float32),
                  bias.astype(jnp.float32)]).reshape(1, 2)
  rating, rating_orig = _run_tc(ue, imean[:NI], imean[NI:], uc, ip, wb)
  return (rating, rating_orig)


# speedup vs baseline: 9.2549x; 1.0002x over previous
import jax
import jax.numpy as jnp
from jax import lax
from jax.experimental import pallas as pl
from jax.experimental.pallas import tpu as pltpu
from jax.experimental.pallas import tpu_sc as plsc

NU = 25000
NI = 25000
NN = NU + NI
D = 64
DH = 32
NLAY = 3
E = 800000
BQ = 1024

NC = 2
NS = 16
CHUNK = 128
BR = 16
NB = 25
RPT = BR * NB
EROWS = RPT * NS
EPAD = EROWS * CHUNK

RZT = 3128
NZ = 25
MCH = 125
NMC = NI // MCH
UPT = BQ // NS


def _sc_body(t0, srcr, dstr, wr, uall,
             t1o, t2o, t3o, imean, uemb,
             acc, ebs, ebd, ebw, rows0, rows1, rows2, rows3, ubuf, us, uidx,
             sem, g0, g1, g2, g3, s0, s1, s2, s3):
  c = lax.axis_index("c")
  s = lax.axis_index("s")
  rbufs = (rows0, rows1, rows2, rows3)
  gsems = (g0, g1, g2, g3)
  ssems = (s0, s1, s2, s3)
  zb = jnp.minimum(s * RZT, NN - RZT)

  def scale(rb, j):
    def grp(g, _):
      wv = ebw[j, pl.ds(g * 16, 16)]
      for k in range(16):
        w = wv[k]
        i = g * 16 + k
        rb[i, :] = rb[i, :] * w
      return 0
    lax.fori_loop(0, CHUNK // 16, grp, 0)

  tins = (t0, t1o, t2o)
  touts = (t1o, t2o, t3o)
  for t in range(NLAY):
    tin = tins[t]
    tout = touts[t]
    def zfill(i, _):
      rows2[i, pl.ds(0, 16)] = jnp.zeros((16,), jnp.float32)
      rows2[i, pl.ds(16, 16)] = jnp.zeros((16,), jnp.float32)
      return 0
    lax.fori_loop(0, CHUNK, zfill, 0)
    for k in range(NZ):
      off = jnp.minimum(zb + k * CHUNK, NN - CHUNK)
      pltpu.async_copy(rows2, acc.at[pl.ds(off, CHUNK)], sem)
    for k in range(NZ):
      off = jnp.minimum(zb + k * CHUNK, NN - CHUNK)
      pltpu.make_async_copy(rows2, acc.at[pl.ds(off, CHUNK)], sem).wait()
    plsc.subcore_barrier()

    def block(b, _):
      row0 = s * RPT + b * BR
      pltpu.async_copy(srcr.at[pl.ds(c * EROWS + row0, BR)], ebs, sem)
      pltpu.async_copy(dstr.at[pl.ds(row0, BR)], ebd, sem)
      pltpu.async_copy(wr.at[pl.ds(row0, BR)], ebw, sem)
      pltpu.make_async_copy(srcr.at[pl.ds(c * EROWS + row0, BR)], ebs, sem).wait()
      pltpu.make_async_copy(dstr.at[pl.ds(row0, BR)], ebd, sem).wait()
      pltpu.make_async_copy(wr.at[pl.ds(row0, BR)], ebw, sem).wait()

      pltpu.async_copy(tin.at[ebs.at[0]], rows0, g0)
      pltpu.async_copy(tin.at[ebs.at[1]], rows1, g1)

      def quad(q, _):
        for k in range(4):
          j = 4 * q + k
          rb = rbufs[k]
          pltpu.make_async_copy(tin.at[ebs.at[j]], rb, gsems[k]).wait()
          kp = (k + 2) % 4
          if k < 2:
            @pl.when(q > 0)
            def _():
              pltpu.make_async_copy(rbufs[kp], acc.at[ebd.at[j]],
                                    ssems[kp]).wait()
            pltpu.async_copy(tin.at[ebs.at[j + 2]], rbufs[kp], gsems[kp])
          else:
            pltpu.make_async_copy(rbufs[kp], acc.at[ebd.at[j]],
                                  ssems[kp]).wait()

            @pl.when(j + 2 < BR)
            def _():
              pltpu.async_copy(tin.at[ebs.at[j + 2]], rbufs[kp], gsems[kp])
          scale(rb, j)
          pltpu.async_copy(rb, acc.at[ebd.at[j]], ssems[k], add=True)
        return 0
      lax.fori_loop(0, BR // 4, quad, 0)
      for k in (2, 3):
        pltpu.make_async_copy(rbufs[k], acc.at[ebd.at[0]], ssems[k]).wait()
      return 0
    lax.fori_loop(0, NB, block, 0)
    plsc.subcore_barrier()
    pltpu.sync_copy(acc.at[pl.ds(zb, RZT)],
                    tout.at[pl.ds(c * NN, NN)].at[pl.ds(zb, RZT)])
    plsc.subcore_barrier()

  def mean_chunk(k, _):
    ck = s + NS * k

    @pl.when(ck < NMC)
    def _():
      base = c * NN + NU + ck * MCH
      for ti, tref in enumerate((t0, t1o, t2o, t3o)):
        pltpu.sync_copy(tref.at[pl.ds(base, MCH)], rows0.at[pl.ds(0, MCH)])

        def row(i, _):
          for h in range(2):
            sl = pl.ds(h * 16, 16)
            if ti == 0:
              rows1[i, sl] = rows0[i, sl]
            elif ti < NLAY:
              rows1[i, sl] = rows1[i, sl] + rows0[i, sl]
            else:
              rows1[i, sl] = (rows1[i, sl] + rows0[i, sl]) * 0.25
          return 0
        lax.fori_loop(0, MCH, row, 0)
      pltpu.sync_copy(rows1.at[pl.ds(0, MCH)],
                      imean.at[pl.ds(c * NI + ck * MCH, MCH)])
    return 0
  lax.fori_loop(0, (NMC + NS - 1) // NS, mean_chunk, 0)

  ub = s * UPT
  pltpu.sync_copy(uall.at[pl.ds(c * BQ + ub, UPT)], uidx)
  for ti, tref in enumerate((t0, t1o, t2o, t3o)):
    pltpu.async_copy(tref.at[uidx], ubuf, sem)
    pltpu.make_async_copy(tref.at[uidx], ubuf, sem).wait()

    def urow(i, _):
      for h in range(2):
        sl = pl.ds(h * 16, 16)
        if ti == 0:
          us[i, sl] = ubuf[i, sl]
        elif ti < NLAY:
          us[i, sl] = us[i, sl] + ubuf[i, sl]
        else:
          us[i, sl] = (us[i, sl] + ubuf[i, sl]) * 0.25
      return 0
    lax.fori_loop(0, UPT, urow, 0)
  pltpu.sync_copy(us, uemb.at[pl.ds(c * BQ + ub, UPT)])


def _run_sc(t0, srcr, dstr, wr, uall):
  mesh = plsc.VectorSubcoreMesh(core_axis_name="c", subcore_axis_name="s",
                                num_cores=NC, num_subcores=NS)
  f = pl.kernel(
      _sc_body,
      out_type=(
          jax.ShapeDtypeStruct((NC * NN, DH), jnp.float32),
          jax.ShapeDtypeStruct((NC * NN, DH), jnp.float32),
          jax.ShapeDtypeStruct((NC * NN, DH), jnp.float32),
          jax.ShapeDtypeStruct((NC * NI, DH), jnp.float32),
          jax.ShapeDtypeStruct((NC * BQ, DH), jnp.float32),
      ),
      mesh=mesh,
      scratch_types=[
          pltpu.VMEM_SHARED((NN, DH), jnp.float32),
          pltpu.VMEM((BR, CHUNK), jnp.int32),
          pltpu.VMEM((BR, CHUNK), jnp.int32),
          pltpu.VMEM((BR, CHUNK), jnp.float32),
          pltpu.VMEM((CHUNK, DH), jnp.float32),
          pltpu.VMEM((CHUNK, DH), jnp.float32),
          pltpu.VMEM((CHUNK, DH), jnp.float32),
          pltpu.VMEM((CHUNK, DH), jnp.float32),
          pltpu.VMEM((UPT, DH), jnp.float32),
          pltpu.VMEM((UPT, DH), jnp.float32),
          pltpu.VMEM((UPT,), jnp.int32),
          pltpu.SemaphoreType.DMA,
          pltpu.SemaphoreType.DMA,
          pltpu.SemaphoreType.DMA,
          pltpu.SemaphoreType.DMA,
          pltpu.SemaphoreType.DMA,
          pltpu.SemaphoreType.DMA,
          pltpu.SemaphoreType.DMA,
          pltpu.SemaphoreType.DMA,
          pltpu.SemaphoreType.DMA,
      ],
      compiler_params=pltpu.CompilerParams(use_tc_tiling_on_sc=False),
  )
  return f(t0, srcr, dstr, wr, uall)


IB = 512
NIB = pl.cdiv(NI, IB)


def _tc_body(ue_ref, ia_ref, ib_ref, uc_ref, ip_ref, wb_ref, rating_ref, orig_ref):
  ue = ue_ref[...]
  dn = (((1,), (1,)), ((), ()))
  logits = lax.dot_general(ue[:, :DH], ia_ref[...], dn,
                           preferred_element_type=jnp.float32)
  logits += lax.dot_general(ue[:, DH:], ib_ref[...], dn,
                            preferred_element_type=jnp.float32)
  orig_ref[...] = jax.nn.sigmoid(logits)
  adj = uc_ref[...] * ip_ref[...] * wb_ref[0, 0] + wb_ref[0, 1]
  rating_ref[...] = jax.nn.sigmoid(logits + adj)


def _run_tc(ue, ia, ib, uc, ip, wb):
  return pl.pallas_call(
      _tc_body,
      grid=(NIB,),
      in_specs=[
          pl.BlockSpec((BQ, D), lambda i: (0, 0)),
          pl.BlockSpec((IB, DH), lambda i: (i, 0)),
          pl.BlockSpec((IB, DH), lambda i: (i, 0)),
          pl.BlockSpec((BQ, 1), lambda i: (0, 0)),
          pl.BlockSpec((1, IB), lambda i: (0, i)),
          pl.BlockSpec(memory_space=pltpu.SMEM),
      ],
      out_specs=[
          pl.BlockSpec((BQ, IB), lambda i: (0, i)),
          pl.BlockSpec((BQ, IB), lambda i: (0, i)),
      ],
      out_shape=[
          jax.ShapeDtypeStruct((BQ, NI), jnp.float32),
          jax.ShapeDtypeStruct((BQ, NI), jnp.float32),
      ],
  )(ue, ia, ib, uc, ip, wb)


def kernel(emb_user, emb_item, edge_weight, user_conf, item_pop, weight, bias,
           edge_index, users):
  allemb = jnp.concatenate([emb_user, emb_item], axis=0)
  t0 = allemb.reshape(NN, NC, DH).transpose(1, 0, 2).reshape(NC * NN, DH)

  src = edge_index[0].astype(jnp.int32)
  dst = edge_index[1].astype(jnp.int32)
  w = edge_weight.astype(jnp.float32)
  pad = EPAD - E
  spread = jnp.arange(pad, dtype=jnp.int32) % NN
  srcp = jnp.concatenate([src, spread])
  srcr = jnp.concatenate([srcp, srcp + NN]).reshape(NC * EROWS, CHUNK)
  dstr = jnp.concatenate([dst, spread]).reshape(EROWS, CHUNK)
  wr = jnp.pad(w, (0, pad)).reshape(EROWS, CHUNK)

  users_i = users.astype(jnp.int32)
  uall = jnp.concatenate([users_i, users_i + NN])

  t1o, t2o, t3o, imean, uemb = _run_sc(t0, srcr, dstr, wr, uall)

  ue = uemb.reshape(NC, BQ, DH).transpose(1, 0, 2).reshape(BQ, D)
  uc = user_conf[users_i, 0].astype(jnp.float32).reshape(BQ, 1)
  ip = item_pop.astype(jnp.float32).reshape(1, NI)
  wb = jnp.stack([weight.astype(jnp.float32),
                  bias.astype(jnp.float32)]).reshape(1, 2)
  rating, rating_orig = _run_tc(ue, imean[:NI], imean[NI:], uc, ip, wb)
  return (rating, rating_orig)


# speedup vs baseline: 9.2618x; 1.0007x over previous
import jax
import jax.numpy as jnp
from jax import lax
from jax.experimental import pallas as pl
from jax.experimental.pallas import tpu as pltpu
from jax.experimental.pallas import tpu_sc as plsc

NU = 25000
NI = 25000
NN = NU + NI
D = 64
DH = 32
NLAY = 3
E = 800000
BQ = 1024

NC = 2
NS = 16
CHUNK = 128
BR = 16
NB = 25
RPT = BR * NB
EROWS = RPT * NS
EPAD = EROWS * CHUNK

RZT = 3128
NZ = 25
MCH = 125
NMC = NI // MCH
UPT = BQ // NS


def _sc_body(t0, srcr, dstr, wr, uall,
             t1o, t2o, t3o, imean, uemb,
             acc, ebs, ebd, ebw, rows0, rows1, rows2, rows3, ubuf, us, uidx,
             sem, g0, g1, g2, g3, s0, s1, s2, s3):
  c = lax.axis_index("c")
  s = lax.axis_index("s")
  rbufs = (rows0, rows1, rows2, rows3)
  gsems = (g0, g1, g2, g3)
  ssems = (s0, s1, s2, s3)
  zb = jnp.minimum(s * RZT, NN - RZT)

  def scale(rb, j):
    def grp(g, _):
      wv = ebw[j, pl.ds(g * 16, 16)]
      for k in range(16):
        w = wv[k]
        i = g * 16 + k
        rb[i, :] = rb[i, :] * w
      return 0
    lax.fori_loop(0, CHUNK // 16, grp, 0)

  tins = (t0, t1o, t2o)
  touts = (t1o, t2o, t3o)
  for t in range(NLAY):
    tin = tins[t]
    tout = touts[t]
    def zfill(i, _):
      rows2[i, pl.ds(0, 16)] = jnp.zeros((16,), jnp.float32)
      rows2[i, pl.ds(16, 16)] = jnp.zeros((16,), jnp.float32)
      return 0
    lax.fori_loop(0, CHUNK, zfill, 0)
    for k in range(NZ):
      off = jnp.minimum(zb + k * CHUNK, NN - CHUNK)
      pltpu.async_copy(rows2, acc.at[pl.ds(off, CHUNK)], sem)
    for k in range(NZ):
      off = jnp.minimum(zb + k * CHUNK, NN - CHUNK)
      pltpu.make_async_copy(rows2, acc.at[pl.ds(off, CHUNK)], sem).wait()
    plsc.subcore_barrier()

    def block(b, _):
      row0 = s * RPT + b * BR
      pltpu.async_copy(srcr.at[pl.ds(row0, BR)], ebs, sem)
      pltpu.async_copy(dstr.at[pl.ds(row0, BR)], ebd, sem)
      pltpu.async_copy(wr.at[pl.ds(row0, BR)], ebw, sem)
      pltpu.make_async_copy(srcr.at[pl.ds(row0, BR)], ebs, sem).wait()
      pltpu.make_async_copy(dstr.at[pl.ds(row0, BR)], ebd, sem).wait()
      pltpu.make_async_copy(wr.at[pl.ds(row0, BR)], ebw, sem).wait()
      coff = c * NN

      def ioff(i, _):
        for h in range(CHUNK // 16):
          sl = pl.ds(h * 16, 16)
          ebs[i, sl] = ebs[i, sl] + coff
        return 0
      lax.fori_loop(0, BR, ioff, 0)

      pltpu.async_copy(tin.at[ebs.at[0]], rows0, g0)
      pltpu.async_copy(tin.at[ebs.at[1]], rows1, g1)

      def quad(q, _):
        for k in range(4):
          j = 4 * q + k
          rb = rbufs[k]
          pltpu.make_async_copy(tin.at[ebs.at[j]], rb, gsems[k]).wait()
          kp = (k + 2) % 4
          if k < 2:
            @pl.when(q > 0)
            def _():
              pltpu.make_async_copy(rbufs[kp], acc.at[ebd.at[j]],
                                    ssems[kp]).wait()
            pltpu.async_copy(tin.at[ebs.at[j + 2]], rbufs[kp], gsems[kp])
          else:
            pltpu.make_async_copy(rbufs[kp], acc.at[ebd.at[j]],
                                  ssems[kp]).wait()

            @pl.when(j + 2 < BR)
            def _():
              pltpu.async_copy(tin.at[ebs.at[j + 2]], rbufs[kp], gsems[kp])
          scale(rb, j)
          pltpu.async_copy(rb, acc.at[ebd.at[j]], ssems[k], add=True)
        return 0
      lax.fori_loop(0, BR // 4, quad, 0)
      for k in (2, 3):
        pltpu.make_async_copy(rbufs[k], acc.at[ebd.at[0]], ssems[k]).wait()
      return 0
    lax.fori_loop(0, NB, block, 0)
    plsc.subcore_barrier()
    pltpu.sync_copy(acc.at[pl.ds(zb, RZT)],
                    tout.at[pl.ds(c * NN, NN)].at[pl.ds(zb, RZT)])
    plsc.subcore_barrier()

  def mean_chunk(k, _):
    ck = s + NS * k

    @pl.when(ck < NMC)
    def _():
      base = c * NN + NU + ck * MCH
      for ti, tref in enumerate((t0, t1o, t2o, t3o)):
        pltpu.sync_copy(tref.at[pl.ds(base, MCH)], rows0.at[pl.ds(0, MCH)])

        def row(i, _):
          for h in range(2):
            sl = pl.ds(h * 16, 16)
            if ti == 0:
              rows1[i, sl] = rows0[i, sl]
            elif ti < NLAY:
              rows1[i, sl] = rows1[i, sl] + rows0[i, sl]
            else:
              rows1[i, sl] = (rows1[i, sl] + rows0[i, sl]) * 0.25
          return 0
        lax.fori_loop(0, MCH, row, 0)
      pltpu.sync_copy(rows1.at[pl.ds(0, MCH)],
                      imean.at[pl.ds(c * NI + ck * MCH, MCH)])
    return 0
  lax.fori_loop(0, (NMC + NS - 1) // NS, mean_chunk, 0)

  ub = s * UPT
  pltpu.sync_copy(uall.at[pl.ds(c * BQ + ub, UPT)], uidx)
  for ti, tref in enumerate((t0, t1o, t2o, t3o)):
    pltpu.async_copy(tref.at[uidx], ubuf, sem)
    pltpu.make_async_copy(tref.at[uidx], ubuf, sem).wait()

    def urow(i, _):
      for h in range(2):
        sl = pl.ds(h * 16, 16)
        if ti == 0:
          us[i, sl] = ubuf[i, sl]
        elif ti < NLAY:
          us[i, sl] = us[i, sl] + ubuf[i, sl]
        else:
          us[i, sl] = (us[i, sl] + ubuf[i, sl]) * 0.25
      return 0
    lax.fori_loop(0, UPT, urow, 0)
  pltpu.sync_copy(us, uemb.at[pl.ds(c * BQ + ub, UPT)])


def _run_sc(t0, srcr, dstr, wr, uall):
  mesh = plsc.VectorSubcoreMesh(core_axis_name="c", subcore_axis_name="s",
                                num_cores=NC, num_subcores=NS)
  f = pl.kernel(
      _sc_body,
      out_type=(
          jax.ShapeDtypeStruct((NC * NN, DH), jnp.float32),
          jax.ShapeDtypeStruct((NC * NN, DH), jnp.float32),
          jax.ShapeDtypeStruct((NC * NN, DH), jnp.float32),
          jax.ShapeDtypeStruct((NC * NI, DH), jnp.float32),
          jax.ShapeDtypeStruct((NC * BQ, DH), jnp.float32),
      ),
      mesh=mesh,
      scratch_types=[
          pltpu.VMEM_SHARED((NN, DH), jnp.float32),
          pltpu.VMEM((BR, CHUNK), jnp.int32),
          pltpu.VMEM((BR, CHUNK), jnp.int32),
          pltpu.VMEM((BR, CHUNK), jnp.float32),
          pltpu.VMEM((CHUNK, DH), jnp.float32),
          pltpu.VMEM((CHUNK, DH), jnp.float32),
          pltpu.VMEM((CHUNK, DH), jnp.float32),
          pltpu.VMEM((CHUNK, DH), jnp.float32),
          pltpu.VMEM((UPT, DH), jnp.float32),
          pltpu.VMEM((UPT, DH), jnp.float32),
          pltpu.VMEM((UPT,), jnp.int32),
          pltpu.SemaphoreType.DMA,
          pltpu.SemaphoreType.DMA,
          pltpu.SemaphoreType.DMA,
          pltpu.SemaphoreType.DMA,
          pltpu.SemaphoreType.DMA,
          pltpu.SemaphoreType.DMA,
          pltpu.SemaphoreType.DMA,
          pltpu.SemaphoreType.DMA,
          pltpu.SemaphoreType.DMA,
      ],
      compiler_params=pltpu.CompilerParams(use_tc_tiling_on_sc=False),
  )
  return f(t0, srcr, dstr, wr, uall)


IB = 512
NIB = pl.cdiv(NI, IB)


def _tc_body(ue_ref, ia_ref, ib_ref, uc_ref, ip_ref, wb_ref, rating_ref, orig_ref):
  ue = ue_ref[...]
  dn = (((1,), (1,)), ((), ()))
  logits = lax.dot_general(ue[:, :DH], ia_ref[...], dn,
                           preferred_element_type=jnp.float32)
  logits += lax.dot_general(ue[:, DH:], ib_ref[...], dn,
                            preferred_element_type=jnp.float32)
  orig_ref[...] = jax.nn.sigmoid(logits)
  adj = uc_ref[...] * ip_ref[...] * wb_ref[0, 0] + wb_ref[0, 1]
  rating_ref[...] = jax.nn.sigmoid(logits + adj)


def _run_tc(ue, ia, ib, uc, ip, wb):
  return pl.pallas_call(
      _tc_body,
      grid=(NIB,),
      in_specs=[
          pl.BlockSpec((BQ, D), lambda i: (0, 0)),
          pl.BlockSpec((IB, DH), lambda i: (i, 0)),
          pl.BlockSpec((IB, DH), lambda i: (i, 0)),
          pl.BlockSpec((BQ, 1), lambda i: (0, 0)),
          pl.BlockSpec((1, IB), lambda i: (0, i)),
          pl.BlockSpec(memory_space=pltpu.SMEM),
      ],
      out_specs=[
          pl.BlockSpec((BQ, IB), lambda i: (0, i)),
          pl.BlockSpec((BQ, IB), lambda i: (0, i)),
      ],
      out_shape=[
          jax.ShapeDtypeStruct((BQ, NI), jnp.float32),
          jax.ShapeDtypeStruct((BQ, NI), jnp.float32),
      ],
  )(ue, ia, ib, uc, ip, wb)


def kernel(emb_user, emb_item, edge_weight, user_conf, item_pop, weight, bias,
           edge_index, users):
  allemb = jnp.concatenate([emb_user, emb_item], axis=0)
  t0 = allemb.reshape(NN, NC, DH).transpose(1, 0, 2).reshape(NC * NN, DH)

  src = edge_index[0].astype(jnp.int32)
  dst = edge_index[1].astype(jnp.int32)
  w = edge_weight.astype(jnp.float32)
  pad = EPAD - E
  spread = jnp.arange(pad, dtype=jnp.int32) % NN
  srcr = jnp.concatenate([src, spread]).reshape(EROWS, CHUNK)
  dstr = jnp.concatenate([dst, spread]).reshape(EROWS, CHUNK)
  wr = jnp.pad(w, (0, pad)).reshape(EROWS, CHUNK)

  users_i = users.astype(jnp.int32)
  uall = jnp.concatenate([users_i, users_i + NN])

  t1o, t2o, t3o, imean, uemb = _run_sc(t0, srcr, dstr, wr, uall)

  ue = uemb.reshape(NC, BQ, DH).transpose(1, 0, 2).reshape(BQ, D)
  uc = user_conf[users_i, 0].astype(jnp.float32).reshape(BQ, 1)
  ip = item_pop.astype(jnp.float32).reshape(1, NI)
  wb = jnp.stack([weight.astype(jnp.float32),
                  bias.astype(jnp.float32)]).reshape(1, 2)
  rating, rating_orig = _run_tc(ue, imean[:NI], imean[NI:], uc, ip, wb)
  return (rating, rating_orig)
